# baseline (speedup 1.0000x reference)
import functools

import jax
import jax.numpy as jnp
from jax import lax
from jax.experimental import pallas as pl
from jax.experimental.pallas import tpu as pltpu
from jax.experimental.pallas import tpu_sc as plsc

_B = 16384
_D = 64
_MARGIN = 1.0
_NC = 2
_NS = 16
_L = 16
_NW = _NC * _NS
_BW = _B // _NW
_C = 128
_NCHUNK = _BW // _C

_mesh = plsc.VectorSubcoreMesh(core_axis_name="c", subcore_axis_name="s")


@functools.partial(
    pl.kernel,
    mesh=_mesh,
    compiler_params=pltpu.CompilerParams(
        needs_layout_passes=False, use_tc_tiling_on_sc=False),
    out_type=jax.ShapeDtypeStruct((_NW, _L), jnp.float32),
    scratch_types=[
        pltpu.VMEM((_C,), jnp.int32),
        pltpu.VMEM((_C,), jnp.int32),
        pltpu.VMEM((_C,), jnp.int32),
        pltpu.VMEM((_C,), jnp.int32),
        pltpu.VMEM((_C,), jnp.int32),
        pltpu.VMEM((_C,), jnp.int32),
        pltpu.VMEM((_C, _D), jnp.float32),
        pltpu.VMEM((_C, _D), jnp.float32),
        pltpu.VMEM((_C, _D), jnp.float32),
        pltpu.VMEM((_C, _D), jnp.float32),
        pltpu.VMEM((_C, _D), jnp.float32),
        pltpu.VMEM((_C, _D), jnp.float32),
        pltpu.VMEM((_L * _L,), jnp.float32),
        pltpu.VMEM((_L,), jnp.float32),
        pltpu.SemaphoreType.DMA,
    ],
)
def _transe_sc(ph, pr, pt, nh, nr, nt, ent, rel, out,
               iph, ipr, ipt, inh, inr, int_,
               rph, rpr, rpt, rnh, rnr, rnt, sbuf, accv, sem):
    wid = lax.axis_index("s") * _NC + lax.axis_index("c")
    base = wid * _BW
    lanes = lax.iota(jnp.int32, _L)

    def chunk(g, acc):
        cb = pl.multiple_of(base + g * _C, _C)
        pltpu.sync_copy(ph.at[pl.ds(cb, _C)], iph)
        pltpu.sync_copy(pr.at[pl.ds(cb, _C)], ipr)
        pltpu.sync_copy(pt.at[pl.ds(cb, _C)], ipt)
        pltpu.sync_copy(nh.at[pl.ds(cb, _C)], inh)
        pltpu.sync_copy(nr.at[pl.ds(cb, _C)], inr)
        pltpu.sync_copy(nt.at[pl.ds(cb, _C)], int_)
        cps = [
            pltpu.async_copy(ent.at[iph], rph, sem),
            pltpu.async_copy(rel.at[ipr], rpr, sem),
            pltpu.async_copy(ent.at[ipt], rpt, sem),
            pltpu.async_copy(ent.at[inh], rnh, sem),
            pltpu.async_copy(rel.at[inr], rnr, sem),
            pltpu.async_copy(ent.at[int_], rnt, sem),
        ]
        for cp in cps:
            cp.wait()

        def group(k, a):
            i0 = k * _L
            for m in range(_L):
                i = i0 + m
                s = jnp.zeros((_L,), jnp.float32)
                for j in range(_D // _L):
                    d = pl.ds(j * _L, _L)
                    dp = jnp.abs(rph[i, d] + rpr[i, d] - rpt[i, d])
                    dn = jnp.abs(rnh[i, d] + rnr[i, d] - rnt[i, d])
                    s = s + (dp - dn)
                sbuf[pl.ds(m * _L, _L)] = s
            rs = jnp.zeros((_L,), jnp.float32)
            for d in range(_L):
                rs = rs + plsc.load_gather(sbuf, [lanes * _L + d])
            return a + jnp.maximum(0.0, _MARGIN + rs)

        return lax.fori_loop(0, _C // _L, group, acc)

    acc = lax.fori_loop(0, _NCHUNK, chunk, jnp.zeros((_L,), jnp.float32))
    accv[...] = acc
    pltpu.sync_copy(accv, out.at[wid])


def kernel(pos_h, pos_r, pos_t, neg_h, neg_r, neg_t, ent_emb, rel_emb):
    parts = _transe_sc(pos_h, pos_r, pos_t, neg_h, neg_r, neg_t,
                       ent_emb, rel_emb)
    return jnp.sum(parts) * (1.0 / _B)


# speedup vs baseline: 1.5739x; 1.5739x over previous
import functools

import jax
import jax.numpy as jnp
from jax import lax
from jax.experimental import pallas as pl
from jax.experimental.pallas import tpu as pltpu
from jax.experimental.pallas import tpu_sc as plsc

_B = 16384
_D = 64
_MARGIN = 1.0
_NC = 2
_NS = 16
_L = 16
_NW = _NC * _NS
_BW = _B // _NW
_C = 128
_NCHUNK = _BW // _C

_mesh = plsc.VectorSubcoreMesh(core_axis_name="c", subcore_axis_name="s")


@functools.partial(
    pl.kernel,
    mesh=_mesh,
    compiler_params=pltpu.CompilerParams(needs_layout_passes=False),
    out_type=jax.ShapeDtypeStruct((_NW, _L), jnp.float32),
    scratch_types=[
        pltpu.VMEM((_C,), jnp.int32),
        pltpu.VMEM((_C,), jnp.int32),
        pltpu.VMEM((_C,), jnp.int32),
        pltpu.VMEM((_C,), jnp.int32),
        pltpu.VMEM((_C,), jnp.int32),
        pltpu.VMEM((_C,), jnp.int32),
        pltpu.VMEM((_C, _D), jnp.float32),
        pltpu.VMEM((_C, _D), jnp.float32),
        pltpu.VMEM((_C, _D), jnp.float32),
        pltpu.VMEM((_C, _D), jnp.float32),
        pltpu.VMEM((_C, _D), jnp.float32),
        pltpu.VMEM((_C, _D), jnp.float32),
        pltpu.VMEM((_L * _L,), jnp.float32),
        pltpu.VMEM((_L,), jnp.float32),
        pltpu.SemaphoreType.DMA,
    ],
)
def _transe_sc(ph, pr, pt, nh, nr, nt, ent, rel, out,
               iph, ipr, ipt, inh, inr, int_,
               rph, rpr, rpt, rnh, rnr, rnt, sbuf, accv, sem):
    wid = lax.axis_index("s") * _NC + lax.axis_index("c")
    base = wid * _BW
    lanes = lax.iota(jnp.int32, _L)

    def chunk(g, acc):
        cb = pl.multiple_of(base + g * _C, _C)
        pltpu.sync_copy(ph.at[pl.ds(cb, _C)], iph)
        pltpu.sync_copy(pr.at[pl.ds(cb, _C)], ipr)
        pltpu.sync_copy(pt.at[pl.ds(cb, _C)], ipt)
        pltpu.sync_copy(nh.at[pl.ds(cb, _C)], inh)
        pltpu.sync_copy(nr.at[pl.ds(cb, _C)], inr)
        pltpu.sync_copy(nt.at[pl.ds(cb, _C)], int_)

        def fire(k, _):
            i0 = k * _L
            vh = iph[pl.ds(i0, _L)]
            vr = ipr[pl.ds(i0, _L)]
            vt = ipt[pl.ds(i0, _L)]
            wh = inh[pl.ds(i0, _L)]
            wr = inr[pl.ds(i0, _L)]
            wt = int_[pl.ds(i0, _L)]
            for m in range(_L):
                i = i0 + m
                pltpu.async_copy(ent.at[vh[m]], rph.at[i], sem)
                pltpu.async_copy(rel.at[vr[m]], rpr.at[i], sem)
                pltpu.async_copy(ent.at[vt[m]], rpt.at[i], sem)
                pltpu.async_copy(ent.at[wh[m]], rnh.at[i], sem)
                pltpu.async_copy(rel.at[wr[m]], rnr.at[i], sem)
                pltpu.async_copy(ent.at[wt[m]], rnt.at[i], sem)
            return 0

        lax.fori_loop(0, _C // _L, fire, 0)
        drain = pltpu.make_async_copy(ent.at[0], rph.at[0], sem)
        for _ in range(6 * _C):
            drain.wait()

        def group(k, a):
            i0 = k * _L
            for m in range(_L):
                i = i0 + m
                s = jnp.zeros((_L,), jnp.float32)
                for j in range(_D // _L):
                    d = pl.ds(j * _L, _L)
                    dp = jnp.abs(rph[i, d] + rpr[i, d] - rpt[i, d])
                    dn = jnp.abs(rnh[i, d] + rnr[i, d] - rnt[i, d])
                    s = s + (dp - dn)
                sbuf[pl.ds(m * _L, _L)] = s
            rs = jnp.zeros((_L,), jnp.float32)
            for d in range(_L):
                rs = rs + plsc.load_gather(sbuf, [lanes * _L + d])
            return a + jnp.maximum(0.0, _MARGIN + rs)

        return lax.fori_loop(0, _C // _L, group, acc)

    acc = lax.fori_loop(0, _NCHUNK, chunk, jnp.zeros((_L,), jnp.float32))
    accv[...] = acc
    pltpu.sync_copy(accv, out.at[wid])


def kernel(pos_h, pos_r, pos_t, neg_h, neg_r, neg_t, ent_emb, rel_emb):
    parts = _transe_sc(pos_h, pos_r, pos_t, neg_h, neg_r, neg_t,
                       ent_emb, rel_emb)
    return jnp.sum(parts) * (1.0 / _B)
